# baseline (speedup 1.0000x reference)
import functools

import jax
import jax.numpy as jnp
from jax import lax
from jax.experimental import pallas as pl
from jax.experimental.pallas import tpu as pltpu
from jax.experimental.pallas import tpu_sc as plsc

VOCAB = 1000000
EMB_DIM = 32
BATCH = 4096
SEQ = 50

_info = plsc.get_sparse_core_info()
NC, NS = _info.num_cores, _info.num_subcores
NW = NC * NS

FAT = 128 // EMB_DIM
N_FAT = VOCAB // FAT
B_PER_W = BATCH // NW
S_CHUNK = 10
N_S_CHUNKS = SEQ // S_CHUNK


def _iota16():
    return lax.broadcasted_iota(jnp.int32, (16,), 0)


def _lookup_kernel(idx_t, fat_hbm, out_hbm,
                   idx_v, c0, c1, fat0, fat1, fidx0, fidx1, gsem, wsem):
    wid = lax.axis_index("s") * NC + lax.axis_index("c")
    b0 = wid * B_PER_W
    cbuf = [c0, c1]
    fat_v = [fat0, fat1]
    fidx = [fidx0, fidx1]
    iota = _iota16()

    pltpu.sync_copy(idx_t.at[:, pl.ds(b0, B_PER_W)], idx_v)

    def prep_and_fire(si, b):
        for c in range(8):
            v = idx_v[si, pl.ds(16 * c, 16)]
            fidx[b][pl.ds(16 * c, 16)] = lax.shift_right_logical(v, 2)
        pltpu.async_copy(fat_hbm.at[fidx[b]], fat_v[b], gsem)

    def extract(si, si_local, b, p):
        for blk in range(8):
            ids = idx_v[si, pl.ds(16 * blk, 16)]
            colb = lax.shift_left(lax.bitwise_and(ids, 3), 5)
            rows = 16 * blk + iota

            def d_body(d, _):
                x = plsc.load_gather(fat_v[b], [rows, colb + d])
                cbuf[p][si_local, d, pl.ds(16 * blk, 16)] = x
                return ()

            lax.fori_loop(0, EMB_DIM, d_body, ())

    prep_and_fire(0, 0)
    for ch in range(N_S_CHUNKS):
        p = ch % 2
        if ch >= 2:
            pltpu.make_async_copy(
                cbuf[p], out_hbm.at[pl.ds(0, S_CHUNK), :, pl.ds(0, B_PER_W)],
                wsem).wait()

        def si_pair(kp, _):
            for b in range(2):
                k = 2 * kp + b
                si = ch * S_CHUNK + k

                @pl.when(si + 1 < SEQ)
                def _():
                    prep_and_fire(si + 1, 1 - b)

                pltpu.make_async_copy(
                    fat_hbm.at[pl.ds(0, B_PER_W)], fat_v[b], gsem).wait()
                extract(si, k, b, p)
            return ()

        lax.fori_loop(0, S_CHUNK // 2, si_pair, ())
        pltpu.async_copy(
            cbuf[p],
            out_hbm.at[pl.ds(ch * S_CHUNK, S_CHUNK), :, pl.ds(b0, B_PER_W)],
            wsem)

    for p in range(2):
        pltpu.make_async_copy(
            cbuf[p], out_hbm.at[pl.ds(0, S_CHUNK), :, pl.ds(0, B_PER_W)],
            wsem).wait()


@jax.jit
def _emb(word_ids, table):
    mesh = plsc.VectorSubcoreMesh(core_axis_name="c", subcore_axis_name="s")
    lookup = functools.partial(
        pl.kernel,
        mesh=mesh,
        out_type=jax.ShapeDtypeStruct((SEQ, EMB_DIM, BATCH), jnp.float32),
        scratch_types=[
            pltpu.VMEM((SEQ, B_PER_W), jnp.int32),
            pltpu.VMEM((S_CHUNK, EMB_DIM, B_PER_W), jnp.float32),
            pltpu.VMEM((S_CHUNK, EMB_DIM, B_PER_W), jnp.float32),
            pltpu.VMEM((B_PER_W, 128), jnp.float32),
            pltpu.VMEM((B_PER_W, 128), jnp.float32),
            pltpu.VMEM((B_PER_W,), jnp.int32),
            pltpu.VMEM((B_PER_W,), jnp.int32),
            pltpu.SemaphoreType.DMA,
            pltpu.SemaphoreType.DMA,
        ],
        compiler_params=pltpu.CompilerParams(needs_layout_passes=False),
    )(_lookup_kernel)

    fat = table.reshape(N_FAT, 128)
    out_phys = lookup(word_ids.T, fat)
    return out_phys.transpose(2, 0, 1)


def kernel(word_ids, table):
    return _emb(word_ids, table)
